# baseline (speedup 1.0000x reference)
import functools

import jax
import jax.numpy as jnp
from jax import lax
from jax.experimental import pallas as pl
from jax.experimental.pallas import tpu as pltpu
from jax.experimental.pallas import tpu_sc as plsc

_EMB = 300
_LANES = 16
_NCHUNK = _EMB // _LANES
_TAIL_OFF = _EMB - _LANES
_BATCH = 25


def _sc_body(table_hbm, idx_hbm, w_hbm, wt_hbm, b_hbm, out_hbm,
             idx_v, row_v, w_v, wt_v, b_v, out_v, sem):
    nc = plsc.get_sparse_core_info().num_cores
    wid = lax.axis_index("s") * nc + lax.axis_index("c")
    pltpu.sync_copy(idx_hbm, idx_v)
    pltpu.sync_copy(w_hbm, w_v)
    pltpu.sync_copy(wt_hbm, wt_v)
    pltpu.sync_copy(b_hbm, b_v)
    row = idx_v[pl.ds(wid, _LANES)][0]
    pltpu.async_copy(table_hbm.at[row], row_v, sem).wait()
    acc = b_v[...]
    for j in range(_NCHUNK):
        acc = acc + row_v[pl.ds(j * _LANES, _LANES)] * w_v[pl.ds(j * _LANES, _LANES)]
    acc = acc + row_v[pl.ds(_TAIL_OFF, _LANES)] * wt_v[...]
    lanes = lax.iota(jnp.int32, _LANES)
    dnums = lax.GatherDimensionNumbers(
        offset_dims=(), collapsed_slice_dims=(0,), start_index_map=(0,))
    for sh in (8, 4, 2, 1):
        perm = lanes ^ sh
        acc = acc + lax.gather(
            acc, perm[:, None], dnums, slice_sizes=(1,),
            mode=lax.GatherScatterMode.PROMISE_IN_BOUNDS)
    out_v[...] = 1.0 / (1.0 + jnp.exp(-acc))
    pltpu.sync_copy(out_v, out_hbm.at[wid])


def kernel(input_words, table, fc_w, fc_b):
    info = plsc.get_sparse_core_info()
    nw = info.num_cores * info.num_subcores

    idx = input_words[:, -1].astype(jnp.int32)
    idx_pad = jnp.zeros((nw + _LANES,), jnp.int32).at[:_BATCH].set(idx)

    w = fc_w.reshape(-1).astype(jnp.float32)
    w_main = w[: _NCHUNK * _LANES]
    w_tail = jnp.zeros((_LANES,), jnp.float32).at[_NCHUNK * _LANES - _TAIL_OFF:].set(
        w[_NCHUNK * _LANES:])
    b_vec = jnp.zeros((_LANES,), jnp.float32).at[0].set(fc_b.reshape(-1)[0].astype(jnp.float32))

    mesh = plsc.VectorSubcoreMesh(core_axis_name="c", subcore_axis_name="s")
    sc_fn = functools.partial(
        pl.kernel,
        mesh=mesh,
        compiler_params=pltpu.CompilerParams(use_tc_tiling_on_sc=True),
        out_type=jax.ShapeDtypeStruct((nw, _LANES), jnp.float32),
        scratch_types=[
            pltpu.VMEM((nw + _LANES,), jnp.int32),
            pltpu.VMEM((_EMB,), jnp.float32),
            pltpu.VMEM((_NCHUNK * _LANES,), jnp.float32),
            pltpu.VMEM((_LANES,), jnp.float32),
            pltpu.VMEM((_LANES,), jnp.float32),
            pltpu.VMEM((_LANES,), jnp.float32),
            pltpu.SemaphoreType.DMA,
        ],
    )(_sc_body)
    out2d = sc_fn(table, idx_pad, w_main, w_tail, b_vec)
    return out2d[:_BATCH, 0]


# speedup vs baseline: 43.0491x; 43.0491x over previous
import functools

import jax
import jax.numpy as jnp
from jax import lax
from jax.experimental import pallas as pl
from jax.experimental.pallas import tpu as pltpu
from jax.experimental.pallas import tpu_sc as plsc

_EMB = 300
_LANES = 16
_TILE = 128
_NCHUNK = _EMB // _LANES
_TAIL_OFF = _EMB - _LANES
_BATCH = 25


def _sc_body(tt_hbm, idx_hbm, w_hbm, wt_hbm, b_hbm, out_hbm,
             idx_v, blk_v, w_v, wt_v, b_v, out_v, sem):
    nc = plsc.get_sparse_core_info().num_cores
    wid = lax.axis_index("s") * nc + lax.axis_index("c")
    pltpu.sync_copy(idx_hbm, idx_v)
    pltpu.sync_copy(w_hbm, w_v)
    pltpu.sync_copy(wt_hbm, wt_v)
    pltpu.sync_copy(b_hbm, b_v)
    row = idx_v[pl.ds(wid, _LANES)][0]
    base = pl.multiple_of((row // _TILE) * _TILE, _TILE)
    off = row - base
    pltpu.async_copy(tt_hbm.at[:, pl.ds(base, _TILE)], blk_v, sem).wait()
    col = jnp.full((_LANES,), off, jnp.int32)
    lanes = lax.iota(jnp.int32, _LANES)
    acc = b_v[...]
    for j in range(_NCHUNK):
        vals = plsc.load_gather(blk_v, [lanes + (j * _LANES), col])
        acc = acc + vals * w_v[pl.ds(j * _LANES, _LANES)]
    tail = plsc.load_gather(blk_v, [lanes + _TAIL_OFF, col])
    acc = acc + tail * wt_v[...]
    dnums = lax.GatherDimensionNumbers(
        offset_dims=(), collapsed_slice_dims=(0,), start_index_map=(0,))
    for sh in (8, 4, 2, 1):
        perm = lanes ^ sh
        acc = acc + lax.gather(
            acc, perm[:, None], dnums, slice_sizes=(1,),
            mode=lax.GatherScatterMode.PROMISE_IN_BOUNDS)
    out_v[...] = 1.0 / (1.0 + jnp.exp(-acc))
    pltpu.sync_copy(out_v, out_hbm.at[wid])


def kernel(input_words, table, fc_w, fc_b):
    info = plsc.get_sparse_core_info()
    nw = info.num_cores * info.num_subcores

    idx = input_words[:, -1].astype(jnp.int32)
    idx_pad = jnp.zeros((nw + _LANES,), jnp.int32).at[:_BATCH].set(idx)

    w = fc_w.reshape(-1).astype(jnp.float32)
    w_main = w[: _NCHUNK * _LANES]
    w_tail = jnp.zeros((_LANES,), jnp.float32).at[_NCHUNK * _LANES - _TAIL_OFF:].set(
        w[_NCHUNK * _LANES:])
    b_vec = jnp.zeros((_LANES,), jnp.float32).at[0].set(fc_b.reshape(-1)[0].astype(jnp.float32))

    mesh = plsc.VectorSubcoreMesh(core_axis_name="c", subcore_axis_name="s")
    sc_fn = functools.partial(
        pl.kernel,
        mesh=mesh,
        compiler_params=pltpu.CompilerParams(needs_layout_passes=False),
        out_type=jax.ShapeDtypeStruct((nw, _LANES), jnp.float32),
        scratch_types=[
            pltpu.VMEM((nw + _LANES,), jnp.int32),
            pltpu.VMEM((_EMB, _TILE), jnp.float32),
            pltpu.VMEM((_NCHUNK * _LANES,), jnp.float32),
            pltpu.VMEM((_LANES,), jnp.float32),
            pltpu.VMEM((_LANES,), jnp.float32),
            pltpu.VMEM((_LANES,), jnp.float32),
            pltpu.SemaphoreType.DMA,
        ],
    )(_sc_body)
    out2d = sc_fn(table.T, idx_pad, w_main, w_tail, b_vec)
    return out2d[:_BATCH, 0]


# speedup vs baseline: 52.4882x; 1.2193x over previous
import functools

import jax
import jax.numpy as jnp
from jax import lax
from jax.experimental import pallas as pl
from jax.experimental.pallas import tpu as pltpu
from jax.experimental.pallas import tpu_sc as plsc

_EMB = 300
_LANES = 16
_TILE = 128
_NCHUNK = _EMB // _LANES
_TAIL_OFF = _EMB - _LANES
_BATCH = 25
_SEQ = 200


def _sc_body(tt_hbm, iw_hbm, w_hbm, b_hbm, out_hbm,
             iw_v, blk_v, w_v, b_v, out_v, sem, wsem):
    nc = plsc.get_sparse_core_info().num_cores
    wid = lax.axis_index("s") * nc + lax.axis_index("c")

    @pl.when(wid < _BATCH)
    def _():
        pltpu.sync_copy(iw_hbm.at[wid], iw_v)
        row = iw_v[pl.ds(_SEQ - _LANES, _LANES)][_LANES - 1]
        base = pl.multiple_of((row // _TILE) * _TILE, _TILE)
        off = row - base
        blk_cp = pltpu.async_copy(tt_hbm.at[:, pl.ds(base, _TILE)], blk_v, sem)
        pltpu.async_copy(w_hbm.at[0], w_v, wsem).wait()
        pltpu.sync_copy(b_hbm, b_v.at[pl.ds(0, 1)])
        blk_cp.wait()
        col = jnp.full((_LANES,), off, jnp.int32)
        lanes = lax.iota(jnp.int32, _LANES)
        acc = jnp.zeros((_LANES,), jnp.float32)
        for j in range(_NCHUNK):
            vals = plsc.load_gather(blk_v, [lanes + (j * _LANES), col])
            acc = acc + vals * w_v[pl.ds(j * _LANES, _LANES)]
        tail = plsc.load_gather(blk_v, [lanes + _TAIL_OFF, col])
        tail_w = jnp.where(lanes >= _NCHUNK * _LANES - _TAIL_OFF,
                           w_v[pl.ds(_TAIL_OFF, _LANES)],
                           jnp.zeros((_LANES,), jnp.float32))
        acc = acc + tail * tail_w
        dnums = lax.GatherDimensionNumbers(
            offset_dims=(), collapsed_slice_dims=(0,), start_index_map=(0,))
        for sh in (8, 4, 2, 1):
            perm = lanes ^ sh
            acc = acc + lax.gather(
                acc, perm[:, None], dnums, slice_sizes=(1,),
                mode=lax.GatherScatterMode.PROMISE_IN_BOUNDS)
        bias = b_v[pl.ds(0, _LANES)][0]
        out_v[...] = 1.0 / (1.0 + jnp.exp(-(acc + bias)))
        pltpu.sync_copy(out_v, out_hbm.at[wid])


def kernel(input_words, table, fc_w, fc_b):
    info = plsc.get_sparse_core_info()
    nw = info.num_cores * info.num_subcores

    mesh = plsc.VectorSubcoreMesh(core_axis_name="c", subcore_axis_name="s")
    sc_fn = functools.partial(
        pl.kernel,
        mesh=mesh,
        compiler_params=pltpu.CompilerParams(needs_layout_passes=False),
        out_type=jax.ShapeDtypeStruct((nw, _LANES), jnp.float32),
        scratch_types=[
            pltpu.VMEM((_SEQ,), jnp.int32),
            pltpu.VMEM((_EMB, _TILE), jnp.float32),
            pltpu.VMEM((_EMB,), jnp.float32),
            pltpu.VMEM((_LANES,), jnp.float32),
            pltpu.VMEM((_LANES,), jnp.float32),
            pltpu.SemaphoreType.DMA,
            pltpu.SemaphoreType.DMA,
        ],
    )(_sc_body)
    out2d = sc_fn(table.T, input_words.astype(jnp.int32),
                  fc_w.astype(jnp.float32), fc_b.astype(jnp.float32))
    return out2d[:_BATCH, 0]
